# Initial kernel scaffold; baseline (speedup 1.0000x reference)
import jax
import jax.numpy as jnp
from jax.experimental import pallas as pl


def kernel(x, indices):
    raise NotImplementedError("write your pallas kernel here")


# speedup vs baseline: 1.0331x; 1.0331x over previous
import jax
import jax.numpy as jnp
from jax import lax
from jax.experimental import pallas as pl
from jax.experimental.pallas import tpu as pltpu
from jax.experimental.pallas import tpu_sc as plsc

B, T, F = 4096, 200, 64
S = 50
ROWS = B * S
NC, NS, L = 2, 16, 16
NW = NC * NS
ROWS_PER_W = ROWS // NW
CHUNK = 800
NCHUNK = ROWS_PER_W // CHUNK
DMA_ROWS = 80
NDMA = CHUNK // DMA_ROWS
VECS = CHUNK // L


def _body(x_hbm, idx_hbm, off_hbm, out_hbm, gidx_v, offs_v, rows_v, sem):
    wid = lax.axis_index("s") * NC + lax.axis_index("c")
    wbase = wid * ROWS_PER_W

    def chunk_body(g, carry):
        base = wbase + g * CHUNK
        pltpu.sync_copy(idx_hbm.at[pl.ds(base, CHUNK)], gidx_v)
        pltpu.sync_copy(off_hbm.at[pl.ds(base, CHUNK)], offs_v)

        for i in range(VECS):
            gidx_v[pl.ds(i * L, L)] = (
                gidx_v[pl.ds(i * L, L)] + offs_v[pl.ds(i * L, L)]
            )

        copies = [
            pltpu.async_copy(
                x_hbm.at[gidx_v.at[pl.ds(j * DMA_ROWS, DMA_ROWS)]],
                rows_v.at[pl.ds(j * DMA_ROWS, DMA_ROWS)],
                sem,
            )
            for j in range(NDMA)
        ]
        for c in copies:
            c.wait()
        pltpu.sync_copy(rows_v, out_hbm.at[pl.ds(base, CHUNK)])
        return carry

    lax.fori_loop(0, NCHUNK, chunk_body, 0)


@jax.jit
def kernel(x, indices):
    x2 = x.reshape(B * T, F)
    idx_flat = indices.reshape(ROWS).astype(jnp.int32)
    offs = (jnp.arange(ROWS, dtype=jnp.int32) // S) * T
    call = pl.kernel(
        _body,
        out_type=jax.ShapeDtypeStruct((ROWS, F), jnp.float32),
        mesh=plsc.VectorSubcoreMesh(core_axis_name="c", subcore_axis_name="s"),
        scratch_types=[
            pltpu.VMEM((CHUNK,), jnp.int32),
            pltpu.VMEM((CHUNK,), jnp.int32),
            pltpu.VMEM((CHUNK, F), jnp.float32),
            pltpu.SemaphoreType.DMA,
        ],
        compiler_params=pltpu.CompilerParams(use_tc_tiling_on_sc=False),
    )
    out = call(x2, idx_flat, offs)
    return out.reshape(B, S, F)


# speedup vs baseline: 1.0460x; 1.0125x over previous
import jax
import jax.numpy as jnp
from jax import lax
from jax.experimental import pallas as pl
from jax.experimental.pallas import tpu as pltpu
from jax.experimental.pallas import tpu_sc as plsc

B, T, F = 4096, 200, 64
S = 50
ROWS = B * S
NC, NS, L = 2, 16, 16
NW = NC * NS
ROWS_PER_W = ROWS // NW
CHUNK = 640
NCHUNK = ROWS_PER_W // CHUNK
DMA_ROWS = 128
NDMA = CHUNK // DMA_ROWS
WVECS = ROWS_PER_W // L


def _body(x_hbm, idx_hbm, off_hbm, out_hbm,
          gidx_v, offs_v, rows0, rows1, sg0, sg1, sw0, sw1):
    wid = lax.axis_index("s") * NC + lax.axis_index("c")
    wbase = wid * ROWS_PER_W

    pltpu.sync_copy(idx_hbm.at[pl.ds(wbase, ROWS_PER_W)], gidx_v)
    pltpu.sync_copy(off_hbm.at[pl.ds(wbase, ROWS_PER_W)], offs_v)
    for i in range(WVECS):
        gidx_v[pl.ds(i * L, L)] = (
            gidx_v[pl.ds(i * L, L)] + offs_v[pl.ds(i * L, L)]
        )

    def fire_gather(g, rows_ref, sem):
        for j in range(NDMA):
            pltpu.async_copy(
                x_hbm.at[gidx_v.at[pl.ds(g * CHUNK + j * DMA_ROWS, DMA_ROWS)]],
                rows_ref.at[pl.ds(j * DMA_ROWS, DMA_ROWS)],
                sem,
            )

    def drain_gather(rows_ref, sem):
        pltpu.make_async_copy(x_hbm.at[pl.ds(0, CHUNK)], rows_ref, sem).wait()

    def fire_write(g, rows_ref, sem):
        pltpu.async_copy(rows_ref, out_hbm.at[pl.ds(wbase + g * CHUNK, CHUNK)], sem)

    def drain_write(rows_ref, sem):
        pltpu.make_async_copy(rows_ref, out_hbm.at[pl.ds(wbase, CHUNK)], sem).wait()

    bufs = ((rows0, sg0, sw0), (rows1, sg1, sw1))

    fire_gather(0, rows0, sg0)
    fire_gather(1, rows1, sg1)

    def lbody(k, carry):
        for b, (rws, sg, sw) in enumerate(bufs):
            g = 2 * k + b
            drain_gather(rws, sg)
            fire_write(g, rws, sw)
            drain_write(rws, sw)
            fire_gather(g + 2, rws, sg)
        return carry

    lax.fori_loop(0, NCHUNK // 2 - 1, lbody, 0)

    for b, (rws, sg, sw) in enumerate(bufs):
        g = NCHUNK - 2 + b
        drain_gather(rws, sg)
        fire_write(g, rws, sw)
        drain_write(rws, sw)


@jax.jit
def kernel(x, indices):
    x2 = x.reshape(B * T, F)
    idx_flat = indices.reshape(ROWS).astype(jnp.int32)
    offs = (jnp.arange(ROWS, dtype=jnp.int32) // S) * T
    call = pl.kernel(
        _body,
        out_type=jax.ShapeDtypeStruct((ROWS, F), jnp.float32),
        mesh=plsc.VectorSubcoreMesh(core_axis_name="c", subcore_axis_name="s"),
        scratch_types=[
            pltpu.VMEM((ROWS_PER_W,), jnp.int32),
            pltpu.VMEM((ROWS_PER_W,), jnp.int32),
            pltpu.VMEM((CHUNK, F), jnp.float32),
            pltpu.VMEM((CHUNK, F), jnp.float32),
            pltpu.SemaphoreType.DMA,
            pltpu.SemaphoreType.DMA,
            pltpu.SemaphoreType.DMA,
            pltpu.SemaphoreType.DMA,
        ],
        compiler_params=pltpu.CompilerParams(use_tc_tiling_on_sc=False),
    )
    out = call(x2, idx_flat, offs)
    return out.reshape(B, S, F)


# speedup vs baseline: 2.8169x; 2.6930x over previous
import jax
import jax.numpy as jnp
from jax import lax
from jax.experimental import pallas as pl
from jax.experimental.pallas import tpu as pltpu
from jax.experimental.pallas import tpu_sc as plsc

B, T, F = 4096, 200, 64
S = 50
NC, NS, L = 2, 16, 16
NW = NC * NS
LT = 128
GROUPS = LT // L


def _body(x_hbm, idx_hbm, out_hbm, idx_v, slab, out_v):
    j = lax.axis_index("s") * NC + lax.axis_index("c")
    iota = lax.iota(jnp.int32, L)

    pltpu.sync_copy(idx_hbm.at[:, pl.ds(j * LT, LT)], idx_v)

    def fc_body(fc, carry):
        pltpu.sync_copy(x_hbm.at[:, fc, pl.ds(j * LT, LT)], slab)
        for s in range(S):
            for g in range(GROUPS):
                tv = idx_v[s, pl.ds(g * L, L)]
                val = plsc.load_gather(slab, [tv, g * L + iota])
                out_v[s, pl.ds(g * L, L)] = val
        pltpu.sync_copy(out_v, out_hbm.at[:, fc, pl.ds(j * LT, LT)])
        return carry

    lax.fori_loop(0, F, fc_body, 0)


@jax.jit
def kernel(x, indices):
    xt = jnp.transpose(x, (1, 2, 0))
    idxt = jnp.transpose(indices.astype(jnp.int32), (1, 0))
    call = pl.kernel(
        _body,
        out_type=jax.ShapeDtypeStruct((S, F, B), jnp.float32),
        mesh=plsc.VectorSubcoreMesh(core_axis_name="c", subcore_axis_name="s"),
        scratch_types=[
            pltpu.VMEM((S, LT), jnp.int32),
            pltpu.VMEM((T, LT), jnp.float32),
            pltpu.VMEM((S, LT), jnp.float32),
        ],
        compiler_params=pltpu.CompilerParams(
            use_tc_tiling_on_sc=True, needs_layout_passes=False
        ),
    )
    outt = call(xt, idxt)
    return jnp.transpose(outt, (2, 0, 1))


# speedup vs baseline: 5.0040x; 1.7764x over previous
import jax
import jax.numpy as jnp
from jax import lax
from jax.experimental import pallas as pl
from jax.experimental.pallas import tpu as pltpu
from jax.experimental.pallas import tpu_sc as plsc

B, T, F = 4096, 200, 64
S = 50
NC, NS, L = 2, 16, 16
NW = NC * NS
LT = 128
GROUPS = LT // L


def _body(x_hbm, idx_hbm, out_hbm,
          idx_v, slab0, slab1, out0, out1, sg0, sg1, sw0, sw1):
    j = lax.axis_index("s") * NC + lax.axis_index("c")
    iota = lax.iota(jnp.int32, L)
    lanes = j * LT

    pltpu.sync_copy(idx_hbm.at[:, pl.ds(lanes, LT)], idx_v)

    def fire_g(fc, slab_ref, sem):
        pltpu.async_copy(x_hbm.at[:, fc, pl.ds(lanes, LT)], slab_ref, sem)

    def drain_g(slab_ref, sem):
        pltpu.make_async_copy(
            x_hbm.at[:, 0, pl.ds(lanes, LT)], slab_ref, sem
        ).wait()

    def fire_w(fc, out_ref, sem):
        pltpu.async_copy(out_ref, out_hbm.at[:, fc, pl.ds(lanes, LT)], sem)

    def drain_w(out_ref, sem):
        pltpu.make_async_copy(
            out_ref, out_hbm.at[:, 0, pl.ds(lanes, LT)], sem
        ).wait()

    def compute(slab_ref, out_ref):
        def s_body(s, carry):
            for g in range(GROUPS):
                tv = idx_v[s, pl.ds(g * L, L)]
                out_ref[s, pl.ds(g * L, L)] = plsc.load_gather(
                    slab_ref, [tv, g * L + iota]
                )
            return carry

        lax.fori_loop(0, S, s_body, 0)

    bufs = ((slab0, out0, sg0, sw0), (slab1, out1, sg1, sw1))

    fire_g(0, slab0, sg0)
    fire_g(1, slab1, sg1)
    for fc in (0, 1):
        slab_r, out_r, sg, sw = bufs[fc]
        drain_g(slab_r, sg)
        compute(slab_r, out_r)
        fire_w(fc, out_r, sw)
        fire_g(fc + 2, slab_r, sg)

    def k_body(k, carry):
        for b, (slab_r, out_r, sg, sw) in enumerate(bufs):
            fc = 2 * k + b
            drain_g(slab_r, sg)
            drain_w(out_r, sw)
            compute(slab_r, out_r)
            fire_w(fc, out_r, sw)
            fire_g(fc + 2, slab_r, sg)
        return carry

    lax.fori_loop(1, F // 2 - 1, k_body, 0)

    for b, (slab_r, out_r, sg, sw) in enumerate(bufs):
        fc = F - 2 + b
        drain_g(slab_r, sg)
        drain_w(out_r, sw)
        compute(slab_r, out_r)
        fire_w(fc, out_r, sw)
    for b, (slab_r, out_r, sg, sw) in enumerate(bufs):
        drain_w(out_r, sw)


@jax.jit
def kernel(x, indices):
    xt = jnp.transpose(x, (1, 2, 0))
    idxt = jnp.transpose(indices.astype(jnp.int32), (1, 0))
    call = pl.kernel(
        _body,
        out_type=jax.ShapeDtypeStruct((S, F, B), jnp.float32),
        mesh=plsc.VectorSubcoreMesh(core_axis_name="c", subcore_axis_name="s"),
        scratch_types=[
            pltpu.VMEM((S, LT), jnp.int32),
            pltpu.VMEM((T, LT), jnp.float32),
            pltpu.VMEM((T, LT), jnp.float32),
            pltpu.VMEM((S, LT), jnp.float32),
            pltpu.VMEM((S, LT), jnp.float32),
            pltpu.SemaphoreType.DMA,
            pltpu.SemaphoreType.DMA,
            pltpu.SemaphoreType.DMA,
            pltpu.SemaphoreType.DMA,
        ],
        compiler_params=pltpu.CompilerParams(
            use_tc_tiling_on_sc=True, needs_layout_passes=False
        ),
    )
    outt = call(xt, idxt)
    return jnp.transpose(outt, (2, 0, 1))


# speedup vs baseline: 5.0184x; 1.0029x over previous
import jax
import jax.numpy as jnp
from jax import lax
from jax.experimental import pallas as pl
from jax.experimental.pallas import tpu as pltpu
from jax.experimental.pallas import tpu_sc as plsc

B, T, F = 4096, 200, 64
S = 50
NC, NS, L = 2, 16, 16
NW = NC * NS
LT = 128
GROUPS = LT // L


def _body(x_hbm, idx_hbm, out_hbm,
          idx_v, slab0, slab1, out0, out1, sg0, sg1, sw0, sw1):
    j = lax.axis_index("s") * NC + lax.axis_index("c")
    iota = lax.iota(jnp.int32, L)
    lanes = j * LT

    pltpu.sync_copy(idx_hbm.at[:, pl.ds(lanes, LT)], idx_v)

    def fire_g(fc, slab_ref, sem):
        h = T // 2
        pltpu.async_copy(
            x_hbm.at[pl.ds(0, h), fc, pl.ds(lanes, LT)],
            slab_ref.at[pl.ds(0, h), :], sem,
        )
        pltpu.async_copy(
            x_hbm.at[pl.ds(h, h), fc, pl.ds(lanes, LT)],
            slab_ref.at[pl.ds(h, h), :], sem,
        )

    def drain_g(slab_ref, sem):
        pltpu.make_async_copy(
            x_hbm.at[:, 0, pl.ds(lanes, LT)], slab_ref, sem
        ).wait()

    def fire_w(fc, out_ref, sem):
        pltpu.async_copy(out_ref, out_hbm.at[:, fc, pl.ds(lanes, LT)], sem)

    def drain_w(out_ref, sem):
        pltpu.make_async_copy(
            out_ref, out_hbm.at[:, 0, pl.ds(lanes, LT)], sem
        ).wait()

    def compute(slab_ref, out_ref):
        def s_body(s, carry):
            for g in range(GROUPS):
                tv = idx_v[s, pl.ds(g * L, L)]
                out_ref[s, pl.ds(g * L, L)] = plsc.load_gather(
                    slab_ref, [tv, g * L + iota]
                )
            return carry

        lax.fori_loop(0, S, s_body, 0)

    bufs = ((slab0, out0, sg0, sw0), (slab1, out1, sg1, sw1))

    fire_g(0, slab0, sg0)
    fire_g(1, slab1, sg1)
    for fc in (0, 1):
        slab_r, out_r, sg, sw = bufs[fc]
        drain_g(slab_r, sg)
        compute(slab_r, out_r)
        fire_w(fc, out_r, sw)
        fire_g(fc + 2, slab_r, sg)

    def k_body(k, carry):
        for b, (slab_r, out_r, sg, sw) in enumerate(bufs):
            fc = 2 * k + b
            drain_g(slab_r, sg)
            drain_w(out_r, sw)
            compute(slab_r, out_r)
            fire_w(fc, out_r, sw)
            fire_g(fc + 2, slab_r, sg)
        return carry

    lax.fori_loop(1, F // 2 - 1, k_body, 0)

    for b, (slab_r, out_r, sg, sw) in enumerate(bufs):
        fc = F - 2 + b
        drain_g(slab_r, sg)
        drain_w(out_r, sw)
        compute(slab_r, out_r)
        fire_w(fc, out_r, sw)
    for b, (slab_r, out_r, sg, sw) in enumerate(bufs):
        drain_w(out_r, sw)


@jax.jit
def kernel(x, indices):
    xt = jnp.transpose(x, (1, 2, 0))
    idxt = jnp.transpose(indices.astype(jnp.int32), (1, 0))
    call = pl.kernel(
        _body,
        out_type=jax.ShapeDtypeStruct((S, F, B), jnp.float32),
        mesh=plsc.VectorSubcoreMesh(core_axis_name="c", subcore_axis_name="s"),
        scratch_types=[
            pltpu.VMEM((S, LT), jnp.int32),
            pltpu.VMEM((T, LT), jnp.float32),
            pltpu.VMEM((T, LT), jnp.float32),
            pltpu.VMEM((S, LT), jnp.float32),
            pltpu.VMEM((S, LT), jnp.float32),
            pltpu.SemaphoreType.DMA,
            pltpu.SemaphoreType.DMA,
            pltpu.SemaphoreType.DMA,
            pltpu.SemaphoreType.DMA,
        ],
        compiler_params=pltpu.CompilerParams(
            use_tc_tiling_on_sc=True, needs_layout_passes=False
        ),
    )
    outt = call(xt, idxt)
    return jnp.transpose(outt, (2, 0, 1))
